# baseline (speedup 1.0000x reference)
import functools
import jax
import jax.numpy as jnp
from jax import lax
from jax.experimental import pallas as pl
from jax.experimental.pallas import tpu as pltpu
from jax.experimental.pallas import tpu_sc as plsc

B = 4; K_DIM = 128; L = 96; S = 2048; KNEI = 16; NHEADS = 4; DH = K_DIM // NHEADS; NL = 2
LL = L * L
BS = B * S
GR = B * LL
NW = 32
RPW = BS // NW
SCALE = 1.0 / (DH ** 0.5)
EPS = 1e-5

_MESH = plsc.VectorSubcoreMesh(core_axis_name="c", subcore_axis_name="s")
_SC_PARAMS = pltpu.CompilerParams(needs_layout_passes=False,
                                  use_tc_tiling_on_sc=False)


def _wid():
    return lax.axis_index("c") * 16 + lax.axis_index("s")


def _sc_gather_rows_body(src_hbm, gidx_hbm, out_hbm, idx_v, row_v, sem):
    w = _wid()
    pltpu.sync_copy(gidx_hbm.at[w], idx_v)
    for c in range(2):
        pltpu.async_copy(src_hbm.at[idx_v.at[c]], row_v, sem).wait()
        pltpu.sync_copy(row_v, out_hbm.at[pl.ds(w * RPW + c * 128, 128)])


def _sc_gather_rows(src, gidx, n_src_rows):
    return pl.kernel(
        _sc_gather_rows_body,
        out_type=jax.ShapeDtypeStruct((BS, K_DIM), jnp.float32),
        mesh=_MESH,
        compiler_params=_SC_PARAMS,
        scratch_types=[
            pltpu.VMEM((2, 128), jnp.int32),
            pltpu.VMEM((128, K_DIM), jnp.float32),
            pltpu.SemaphoreType.DMA,
        ],
    )(src, gidx)


def _sc_attn_body(qt, kf, vf, nidx_hbm, wres_hbm, o_hbm, ores_hbm,
                  tab, idxb, scT, qob, obuf, widx_v, rbuf, sem):
    cid = lax.axis_index("c")
    sid = lax.axis_index("s")
    b = 2 * cid + sid // 8
    u = sid % 8
    h = u // 2
    half = u % 2
    qbase0 = b * S + half * 1024
    col0 = 32 * h
    iota = lax.iota(jnp.int32, 16)

    def fc(v):
        return jnp.full((16,), v, jnp.int32)

    pltpu.sync_copy(nidx_hbm.at[pl.ds(qbase0, 1024)], idxb)

    pltpu.sync_copy(kf.at[pl.ds(b * S, S), pl.ds(col0, DH)], tab)

    def chunk_a(c, _):
        pltpu.sync_copy(qt.at[pl.ds(col0, DH), pl.ds(qbase0 + c * 128, 128)], qob)

        def group_a(g, _):
            srow = c * 128 + g * 16
            ni = [plsc.load_gather(idxb, [iota + srow, fc(k)]) for k in range(KNEI)]
            acc = [None] * KNEI
            for d in range(DH):
                qd = qob[d, pl.ds(g * 16, 16)]
                for k in range(KNEI):
                    t = qd * plsc.load_gather(tab, [ni[k], fc(d)])
                    acc[k] = t if acc[k] is None else acc[k] + t
            for k in range(KNEI):
                scT[k, pl.ds(srow, 16)] = acc[k] * SCALE
            return 0

        lax.fori_loop(0, 8, group_a, 0)
        return 0

    lax.fori_loop(0, 8, chunk_a, 0)

    pltpu.sync_copy(vf.at[pl.ds(b * S, S), pl.ds(col0, DH)], tab)

    def chunk_b(c, _):
        def group_b(g, _):
            srow = c * 128 + g * 16
            sc = [scT[k, pl.ds(srow, 16)] for k in range(KNEI)]
            m = sc[0]
            for k in range(1, KNEI):
                m = jnp.maximum(m, sc[k])
            ek = [jnp.exp(sc[k] - m) for k in range(KNEI)]
            den = ek[0]
            for k in range(1, KNEI):
                den = den + ek[k]
            inv = 1.0 / den
            wk = [ek[k] * inv for k in range(KNEI)]
            ni = [plsc.load_gather(idxb, [iota + srow, fc(k)]) for k in range(KNEI)]
            orow = iota + g * 16
            for db in range(4):
                od = [None] * 8
                for k in range(KNEI):
                    for dj in range(8):
                        d = db * 8 + dj
                        t = wk[k] * plsc.load_gather(tab, [ni[k], fc(d)])
                        od[dj] = t if od[dj] is None else od[dj] + t
                for dj in range(8):
                    plsc.store_scatter(obuf, [orow, fc(db * 8 + dj)], od[dj])
            return 0

        lax.fori_loop(0, 8, group_b, 0)
        pltpu.sync_copy(obuf, o_hbm.at[pl.ds(qbase0 + c * 128, 128), pl.ds(col0, DH)])
        return 0

    lax.fori_loop(0, 8, chunk_b, 0)

    plsc.subcore_barrier()
    w = _wid()
    pltpu.sync_copy(wres_hbm.at[w], widx_v)
    for c in range(2):
        pltpu.async_copy(o_hbm.at[widx_v.at[c]], rbuf, sem).wait()
        pltpu.sync_copy(rbuf, ores_hbm.at[pl.ds(w * RPW + c * 128, 128)])


def _sc_attn(qf, kf, vf, nidx, wres):
    return pl.kernel(
        _sc_attn_body,
        out_type=(jax.ShapeDtypeStruct((BS, K_DIM), jnp.float32),
                  jax.ShapeDtypeStruct((BS, K_DIM), jnp.float32)),
        mesh=_MESH,
        compiler_params=_SC_PARAMS,
        scratch_types=[
            pltpu.VMEM((S, DH), jnp.float32),
            pltpu.VMEM((1024, KNEI), jnp.int32),
            pltpu.VMEM((KNEI, 1024), jnp.float32),
            pltpu.VMEM((DH, 128), jnp.float32),
            pltpu.VMEM((128, DH), jnp.float32),
            pltpu.VMEM((2, 128), jnp.int32),
            pltpu.VMEM((128, K_DIM), jnp.float32),
            pltpu.SemaphoreType.DMA,
        ],
    )(qf, kf, vf, nidx, wres)


def _sc_assemble_body(ugrid, rows_f, sidx_hbm, out_hbm, big_v, idx_v, row_v, sem):
    cid = lax.axis_index("c")
    sid = lax.axis_index("s")
    copy_base = cid * (GR // 2) + sid * (GR // NW)
    for c in range(2):
        off = copy_base + c * 576
        pltpu.sync_copy(ugrid.at[pl.ds(off, 576)], big_v)
        pltpu.sync_copy(big_v, out_hbm.at[pl.ds(off, 576)])
    plsc.subcore_barrier()
    w = cid * 16 + sid
    pltpu.sync_copy(sidx_hbm.at[w], idx_v)
    for c in range(2):
        pltpu.sync_copy(rows_f.at[pl.ds(w * RPW + c * 128, 128)], row_v)
        pltpu.async_copy(row_v, out_hbm.at[idx_v.at[c]], sem).wait()


def _sc_assemble(ugrid, rows_f, sidx):
    return pl.kernel(
        _sc_assemble_body,
        out_type=jax.ShapeDtypeStruct((GR, K_DIM), jnp.float32),
        mesh=_MESH,
        compiler_params=_SC_PARAMS,
        scratch_types=[
            pltpu.VMEM((576, K_DIM), jnp.float32),
            pltpu.VMEM((2, 128), jnp.int32),
            pltpu.VMEM((128, K_DIM), jnp.float32),
            pltpu.SemaphoreType.DMA,
        ],
    )(ugrid, rows_f, sidx)


def _ln_rows(x, s_ref, b_ref):
    mu = jnp.mean(x, axis=-1, keepdims=True)
    var = jnp.mean((x - mu) ** 2, axis=-1, keepdims=True)
    return (x - mu) * lax.rsqrt(var + EPS) * s_ref[...] + b_ref[...]


def _matT(x, w_ref):
    return lax.dot_general(x, w_ref[...], (((1,), (1,)), ((), ())),
                           preferred_element_type=jnp.float32)


def _tc_qkv1_body(h_ref, lis, lib, wq, wk, wv, q_ref, k_ref, v_ref):
    hn = _ln_rows(h_ref[...], lis, lib)
    q_ref[...] = _matT(hn, wq).T
    k_ref[...] = _matT(hn, wk)
    v_ref[...] = _matT(hn, wv)


def _tc_upd_qkv_body(prev_ref, ores_ref, wo, lcs, lcb, lis, lib, wq, wk, wv,
                     rows_ref, q_ref, k_ref, v_ref):
    r2 = _ln_rows(prev_ref[...] + _matT(ores_ref[...], wo), lcs, lcb)
    rows_ref[...] = r2
    hn = _ln_rows(r2, lis, lib)
    q_ref[...] = _matT(hn, wq).T
    k_ref[...] = _matT(hn, wk)
    v_ref[...] = _matT(hn, wv)


def _tc_upd_body(prev_ref, ores_ref, wo, lcs, lcb, rows_ref):
    rows_ref[...] = _ln_rows(prev_ref[...] + _matT(ores_ref[...], wo), lcs, lcb)


def _tc_gridln_body(x_ref, s1, b1, s2, b2, o_ref):
    o_ref[...] = _ln_rows(_ln_rows(x_ref[...], s1, b1), s2, b2)


def _rows_spec(blk):
    return pl.BlockSpec((blk, K_DIM), lambda i: (i, 0))


def _full_spec(shape):
    return pl.BlockSpec(shape, lambda i: tuple(0 for _ in shape))


_VEC = _full_spec((1, K_DIM))
_WMAT = _full_spec((K_DIM, K_DIM))
_ROWS_T = jax.ShapeDtypeStruct((BS, K_DIM), jnp.float32)
_QT_T = jax.ShapeDtypeStruct((K_DIM, BS), jnp.float32)


def _qt_spec():
    return pl.BlockSpec((K_DIM, 1024), lambda i: (0, i))


def _tc_qkv1(h, lis, lib, wq, wk, wv):
    return pl.pallas_call(
        _tc_qkv1_body, grid=(8,),
        in_specs=[_rows_spec(1024), _VEC, _VEC, _WMAT, _WMAT, _WMAT],
        out_specs=[_qt_spec(), _rows_spec(1024), _rows_spec(1024)],
        out_shape=[_QT_T, _ROWS_T, _ROWS_T],
    )(h, lis, lib, wq, wk, wv)


def _tc_upd_qkv(prev, ores, wo, lcs, lcb, lis, lib, wq, wk, wv):
    return pl.pallas_call(
        _tc_upd_qkv_body, grid=(8,),
        in_specs=[_rows_spec(1024), _rows_spec(1024), _WMAT, _VEC, _VEC,
                  _VEC, _VEC, _WMAT, _WMAT, _WMAT],
        out_specs=[_rows_spec(1024), _qt_spec(), _rows_spec(1024), _rows_spec(1024)],
        out_shape=[_ROWS_T, _QT_T, _ROWS_T, _ROWS_T],
    )(prev, ores, wo, lcs, lcb, lis, lib, wq, wk, wv)


def _tc_upd(prev, ores, wo, lcs, lcb):
    return pl.pallas_call(
        _tc_upd_body, grid=(8,),
        in_specs=[_rows_spec(1024), _rows_spec(1024), _WMAT, _VEC, _VEC],
        out_specs=_rows_spec(1024),
        out_shape=_ROWS_T,
    )(prev, ores, wo, lcs, lcb)


def _tc_gridln(gf, s1, b1, s2, b2):
    return pl.pallas_call(
        _tc_gridln_body, grid=(9,),
        in_specs=[_rows_spec(4096), _VEC, _VEC, _VEC, _VEC],
        out_specs=_rows_spec(4096),
        out_shape=jax.ShapeDtypeStruct((GR, K_DIM), jnp.float32),
    )(gf, s1, b1, s2, b2)


def kernel(grid_scores, N_idx, N_mask, id2lr_pad, S_row_mask, Wq, Wk, Wv, Wo,
           ln_in_scale, ln_in_bias, ln_ch_scale, ln_ch_bias):
    f32 = jnp.float32
    lin = jnp.clip(id2lr_pad[..., 0] * L + id2lr_pad[..., 1], 0, LL - 1).astype(jnp.int32)
    barange = jnp.broadcast_to(jnp.arange(S, dtype=jnp.int32)[None], (B, S))
    wcell = jnp.zeros((B, LL), jnp.int32).at[jnp.arange(B)[:, None], lin].set(barange)
    w_s = jnp.take_along_axis(wcell, lin, axis=1)
    n32 = jnp.clip(N_idx, 0, S - 1).astype(jnp.int32).reshape(BS, KNEI)
    gidx = (jnp.arange(B, dtype=jnp.int32)[:, None] * LL + lin).reshape(NW, 2, 128)
    wres = (jnp.arange(B, dtype=jnp.int32)[:, None] * S + w_s).reshape(NW, 2, 128)

    gf = grid_scores.reshape(B, K_DIM, LL).transpose(0, 2, 1).reshape(GR, K_DIM)
    lv = [a.reshape(NL, 1, K_DIM).astype(f32)
          for a in (ln_in_scale, ln_in_bias, ln_ch_scale, ln_ch_bias)]
    lis, lib, lcs, lcb = lv

    rows = _sc_gather_rows(gf, gidx, GR)

    q1, k1, v1 = _tc_qkv1(rows, lis[0], lib[0], Wq[0], Wk[0], Wv[0])
    _, ores1 = _sc_attn(q1, k1, v1, n32, wres)
    rows2, q2, k2, v2 = _tc_upd_qkv(rows, ores1, Wo[0], lcs[0], lcb[0],
                                    lis[1], lib[1], Wq[1], Wk[1], Wv[1])
    _, ores2 = _sc_attn(q2, k2, v2, n32, wres)
    rows_f = _tc_upd(rows2, ores2, Wo[1], lcs[1], lcb[1])

    ugrid = _tc_gridln(gf, lcs[0], lcb[0], lcs[1], lcb[1])
    out = _sc_assemble(ugrid, rows_f, gidx)

    return out.reshape(B, LL, K_DIM).transpose(0, 2, 1).reshape(B, K_DIM, L, L)


# speedup vs baseline: 2.3247x; 2.3247x over previous
import functools
import jax
import jax.numpy as jnp
from jax import lax
from jax.experimental import pallas as pl
from jax.experimental.pallas import tpu as pltpu
from jax.experimental.pallas import tpu_sc as plsc

B = 4; K_DIM = 128; L = 96; S = 2048; KNEI = 16; NHEADS = 4; DH = K_DIM // NHEADS; NL = 2
LL = L * L
BS = B * S
GR = B * LL
NW = 32
RPW = BS // NW
HP = DH // 2
SCALE = 1.0 / (DH ** 0.5)
EPS = 1e-5

_MESH = plsc.VectorSubcoreMesh(core_axis_name="c", subcore_axis_name="s")
_SC_PARAMS = pltpu.CompilerParams(needs_layout_passes=False,
                                  use_tc_tiling_on_sc=False)


def _wid():
    return lax.axis_index("c") * 16 + lax.axis_index("s")


def _sc_gather_rows_body(src_hbm, gidx_hbm, out_hbm, idx_v, row_v, sem):
    w = _wid()
    pltpu.sync_copy(gidx_hbm.at[w], idx_v)
    for c in range(2):
        pltpu.async_copy(src_hbm.at[idx_v.at[c]], row_v, sem).wait()
        pltpu.sync_copy(row_v, out_hbm.at[pl.ds(w * RPW + c * 128, 128)])


def _sc_gather_rows(src, gidx, n_src_rows):
    return pl.kernel(
        _sc_gather_rows_body,
        out_type=jax.ShapeDtypeStruct((BS, K_DIM), jnp.float32),
        mesh=_MESH,
        compiler_params=_SC_PARAMS,
        scratch_types=[
            pltpu.VMEM((2, 128), jnp.int32),
            pltpu.VMEM((128, K_DIM), jnp.float32),
            pltpu.SemaphoreType.DMA,
        ],
    )(src, gidx)


def _sc_attn_body(qt, kf, vf, nidx_hbm, wres_hbm, o_hbm, ores_hbm,
                  tab, idxb, scT, qob, obuf, widx_v, rbuf, sem):
    cid = lax.axis_index("c")
    sid = lax.axis_index("s")
    b = 2 * cid + sid // 8
    u = sid % 8
    h = u // 2
    half = u % 2
    qbase0 = b * S + half * 1024
    col0 = 32 * h
    iota = lax.iota(jnp.int32, 16)

    def fc(v):
        return jnp.full((16,), v, jnp.int32)

    pltpu.sync_copy(nidx_hbm.at[:, pl.ds(qbase0, 1024)], idxb)

    hp0 = 16 * h
    pltpu.sync_copy(kf.at[pl.ds(b * S, S), pl.ds(hp0, HP)], tab.at[:, pl.ds(0, HP)])

    def chunk_a(c, _):
        pltpu.sync_copy(qt.at[pl.ds(col0, DH), pl.ds(qbase0 + c * 128, 128)], qob)

        def group_a(g, _):
            srow = c * 128 + g * 16
            ni = [idxb[k, pl.ds(srow, 16)] for k in range(KNEI)]
            acc = [None] * KNEI
            for j in range(HP):
                qa = qob[2 * j, pl.ds(g * 16, 16)]
                qb = qob[2 * j + 1, pl.ds(g * 16, 16)]
                for k in range(KNEI):
                    wd = plsc.load_gather(tab, [ni[k], fc(j)])
                    ea, eb = plsc.unpack(plsc.bitcast(wd, jnp.bfloat16),
                                         format=plsc.PackFormat.INTERLEAVED)
                    t = qa * ea + qb * eb
                    acc[k] = t if acc[k] is None else acc[k] + t
            for k in range(KNEI):
                scT[k, pl.ds(srow, 16)] = acc[k] * SCALE
            return 0

        lax.fori_loop(0, 8, group_a, 0)
        return 0

    lax.fori_loop(0, 8, chunk_a, 0)

    pltpu.sync_copy(vf.at[pl.ds(b * S, S), pl.ds(hp0, HP)], tab.at[:, pl.ds(0, HP)])

    def chunk_b(c, _):
        def group_b(g, _):
            srow = c * 128 + g * 16
            sc = [scT[k, pl.ds(srow, 16)] for k in range(KNEI)]
            m = sc[0]
            for k in range(1, KNEI):
                m = jnp.maximum(m, sc[k])
            ek = [jnp.exp(sc[k] - m) for k in range(KNEI)]
            den = ek[0]
            for k in range(1, KNEI):
                den = den + ek[k]
            inv = 1.0 / den
            wk = [ek[k] * inv for k in range(KNEI)]
            ni = [idxb[k, pl.ds(srow, 16)] for k in range(KNEI)]
            orow = iota + g * 16
            for jb in range(4):
                od = [None] * 8
                for k in range(KNEI):
                    for ji in range(4):
                        j = jb * 4 + ji
                        wd = plsc.load_gather(tab, [ni[k], fc(j)])
                        ea, eb = plsc.unpack(plsc.bitcast(wd, jnp.bfloat16),
                                             format=plsc.PackFormat.INTERLEAVED)
                        t0 = wk[k] * ea
                        t1 = wk[k] * eb
                        od[2 * ji] = t0 if od[2 * ji] is None else od[2 * ji] + t0
                        od[2 * ji + 1] = t1 if od[2 * ji + 1] is None else od[2 * ji + 1] + t1
                for ji in range(4):
                    j = jb * 4 + ji
                    plsc.store_scatter(obuf, [orow, fc(2 * j)], od[2 * ji])
                    plsc.store_scatter(obuf, [orow, fc(2 * j + 1)], od[2 * ji + 1])
            return 0

        lax.fori_loop(0, 8, group_b, 0)
        pltpu.sync_copy(obuf.at[:, pl.ds(0, DH)],
                        o_hbm.at[pl.ds(qbase0 + c * 128, 128), pl.ds(col0, DH)])
        return 0

    lax.fori_loop(0, 8, chunk_b, 0)

    plsc.subcore_barrier()
    w = _wid()
    pltpu.sync_copy(wres_hbm.at[w], widx_v)
    for c in range(8):
        pltpu.async_copy(o_hbm.at[widx_v.at[c]], rbuf, sem).wait()
        pltpu.sync_copy(rbuf, ores_hbm.at[pl.ds(w * RPW + c * 32, 32)])


def _sc_attn(qf, kf, vf, nidx, wres):
    return pl.kernel(
        _sc_attn_body,
        out_type=(jax.ShapeDtypeStruct((BS, K_DIM), jnp.float32),
                  jax.ShapeDtypeStruct((BS, K_DIM), jnp.float32)),
        mesh=_MESH,
        compiler_params=_SC_PARAMS,
        scratch_types=[
            pltpu.VMEM((S, HP + 1), jnp.int32),
            pltpu.VMEM((KNEI, 1024), jnp.int32),
            pltpu.VMEM((KNEI, 1024), jnp.float32),
            pltpu.VMEM((DH, 128), jnp.float32),
            pltpu.VMEM((128, DH + 1), jnp.float32),
            pltpu.VMEM((8, 32), jnp.int32),
            pltpu.VMEM((32, K_DIM), jnp.float32),
            pltpu.SemaphoreType.DMA,
        ],
    )(qf, kf, vf, nidx, wres)


def _sc_assemble_body(ugrid, rows_f, sidx_hbm, out_hbm, big_v, idx_v, row_v, sem):
    cid = lax.axis_index("c")
    sid = lax.axis_index("s")
    copy_base = cid * (GR // 2) + sid * (GR // NW)
    for c in range(2):
        off = copy_base + c * 576
        pltpu.sync_copy(ugrid.at[pl.ds(off, 576)], big_v)
        pltpu.sync_copy(big_v, out_hbm.at[pl.ds(off, 576)])
    plsc.subcore_barrier()
    w = cid * 16 + sid
    pltpu.sync_copy(sidx_hbm.at[w], idx_v)
    for c in range(2):
        pltpu.sync_copy(rows_f.at[pl.ds(w * RPW + c * 128, 128)], row_v)
        pltpu.async_copy(row_v, out_hbm.at[idx_v.at[c]], sem).wait()


def _sc_assemble(ugrid, rows_f, sidx):
    return pl.kernel(
        _sc_assemble_body,
        out_type=jax.ShapeDtypeStruct((GR, K_DIM), jnp.float32),
        mesh=_MESH,
        compiler_params=_SC_PARAMS,
        scratch_types=[
            pltpu.VMEM((576, K_DIM), jnp.float32),
            pltpu.VMEM((2, 128), jnp.int32),
            pltpu.VMEM((128, K_DIM), jnp.float32),
            pltpu.SemaphoreType.DMA,
        ],
    )(ugrid, rows_f, sidx)


def _ln_rows(x, s_ref, b_ref):
    mu = jnp.mean(x, axis=-1, keepdims=True)
    var = jnp.mean((x - mu) ** 2, axis=-1, keepdims=True)
    return (x - mu) * lax.rsqrt(var + EPS) * s_ref[...] + b_ref[...]


def _matT(x, w_ref):
    return lax.dot_general(x, w_ref[...], (((1,), (1,)), ((), ())),
                           preferred_element_type=jnp.float32)


def _tc_qkv1_body(h_ref, lis, lib, wq, wk, wv, q_ref, k_ref, v_ref):
    hn = _ln_rows(h_ref[...], lis, lib)
    q_ref[...] = _matT(hn, wq).T
    k_ref[...] = _matT(hn, wk)
    v_ref[...] = _matT(hn, wv)


def _tc_upd_qkv_body(prev_ref, ores_ref, wo, lcs, lcb, lis, lib, wq, wk, wv,
                     rows_ref, q_ref, k_ref, v_ref):
    r2 = _ln_rows(prev_ref[...] + _matT(ores_ref[...], wo), lcs, lcb)
    rows_ref[...] = r2
    hn = _ln_rows(r2, lis, lib)
    q_ref[...] = _matT(hn, wq).T
    k_ref[...] = _matT(hn, wk)
    v_ref[...] = _matT(hn, wv)


def _tc_upd_body(prev_ref, ores_ref, wo, lcs, lcb, rows_ref):
    rows_ref[...] = _ln_rows(prev_ref[...] + _matT(ores_ref[...], wo), lcs, lcb)


def _tc_gridln_body(x_ref, s1, b1, s2, b2, o_ref):
    o_ref[...] = _ln_rows(_ln_rows(x_ref[...], s1, b1), s2, b2)


def _rows_spec(blk):
    return pl.BlockSpec((blk, K_DIM), lambda i: (i, 0))


def _full_spec(shape):
    return pl.BlockSpec(shape, lambda i: tuple(0 for _ in shape))


_VEC = _full_spec((1, K_DIM))
_WMAT = _full_spec((K_DIM, K_DIM))
_ROWS_T = jax.ShapeDtypeStruct((BS, K_DIM), jnp.float32)
_QT_T = jax.ShapeDtypeStruct((K_DIM, BS), jnp.float32)


def _qt_spec():
    return pl.BlockSpec((K_DIM, 1024), lambda i: (0, i))


def _tc_qkv1(h, lis, lib, wq, wk, wv):
    return pl.pallas_call(
        _tc_qkv1_body, grid=(8,),
        in_specs=[_rows_spec(1024), _VEC, _VEC, _WMAT, _WMAT, _WMAT],
        out_specs=[_qt_spec(), _rows_spec(1024), _rows_spec(1024)],
        out_shape=[_QT_T, _ROWS_T, _ROWS_T],
    )(h, lis, lib, wq, wk, wv)


def _tc_upd_qkv(prev, ores, wo, lcs, lcb, lis, lib, wq, wk, wv):
    return pl.pallas_call(
        _tc_upd_qkv_body, grid=(8,),
        in_specs=[_rows_spec(1024), _rows_spec(1024), _WMAT, _VEC, _VEC,
                  _VEC, _VEC, _WMAT, _WMAT, _WMAT],
        out_specs=[_rows_spec(1024), _qt_spec(), _rows_spec(1024), _rows_spec(1024)],
        out_shape=[_ROWS_T, _QT_T, _ROWS_T, _ROWS_T],
    )(prev, ores, wo, lcs, lcb, lis, lib, wq, wk, wv)


def _tc_upd(prev, ores, wo, lcs, lcb):
    return pl.pallas_call(
        _tc_upd_body, grid=(8,),
        in_specs=[_rows_spec(1024), _rows_spec(1024), _WMAT, _VEC, _VEC],
        out_specs=_rows_spec(1024),
        out_shape=_ROWS_T,
    )(prev, ores, wo, lcs, lcb)


def _tc_gridln(gf, s1, b1, s2, b2):
    return pl.pallas_call(
        _tc_gridln_body, grid=(9,),
        in_specs=[_rows_spec(4096), _VEC, _VEC, _VEC, _VEC],
        out_specs=_rows_spec(4096),
        out_shape=jax.ShapeDtypeStruct((GR, K_DIM), jnp.float32),
    )(gf, s1, b1, s2, b2)


def kernel(grid_scores, N_idx, N_mask, id2lr_pad, S_row_mask, Wq, Wk, Wv, Wo,
           ln_in_scale, ln_in_bias, ln_ch_scale, ln_ch_bias):
    f32 = jnp.float32
    lin = jnp.clip(id2lr_pad[..., 0] * L + id2lr_pad[..., 1], 0, LL - 1).astype(jnp.int32)
    barange = jnp.broadcast_to(jnp.arange(S, dtype=jnp.int32)[None], (B, S))
    wcell = jnp.zeros((B, LL), jnp.int32).at[jnp.arange(B)[:, None], lin].set(barange)
    w_s = jnp.take_along_axis(wcell, lin, axis=1)
    n32 = jnp.clip(N_idx, 0, S - 1).astype(jnp.int32).reshape(BS, KNEI).T
    gidx = (jnp.arange(B, dtype=jnp.int32)[:, None] * LL + lin).reshape(NW, 2, 128)
    wres = (jnp.arange(B, dtype=jnp.int32)[:, None] * S + w_s).reshape(NW, 8, 32)

    gf = grid_scores.reshape(B, K_DIM, LL).transpose(0, 2, 1).reshape(GR, K_DIM)
    lv = [a.reshape(NL, 1, K_DIM).astype(f32)
          for a in (ln_in_scale, ln_in_bias, ln_ch_scale, ln_ch_bias)]
    lis, lib, lcs, lcb = lv

    rows = _sc_gather_rows(gf, gidx, GR)

    def packpairs(x):
        return jax.lax.bitcast_convert_type(
            x.astype(jnp.bfloat16).reshape(BS, K_DIM // 2, 2), jnp.int32)

    q1, k1, v1 = _tc_qkv1(rows, lis[0], lib[0], Wq[0], Wk[0], Wv[0])
    _, ores1 = _sc_attn(q1, packpairs(k1), packpairs(v1), n32, wres)
    rows2, q2, k2, v2 = _tc_upd_qkv(rows, ores1, Wo[0], lcs[0], lcb[0],
                                    lis[1], lib[1], Wq[1], Wk[1], Wv[1])
    _, ores2 = _sc_attn(q2, packpairs(k2), packpairs(v2), n32, wres)
    rows_f = _tc_upd(rows2, ores2, Wo[1], lcs[1], lcb[1])

    ugrid = _tc_gridln(gf, lcs[0], lcb[0], lcs[1], lcb[1])
    out = _sc_assemble(ugrid, rows_f, gidx)

    return out.reshape(B, LL, K_DIM).transpose(0, 2, 1).reshape(B, K_DIM, L, L)


# speedup vs baseline: 2.3394x; 1.0063x over previous
import functools
import jax
import jax.numpy as jnp
from jax import lax
from jax.experimental import pallas as pl
from jax.experimental.pallas import tpu as pltpu
from jax.experimental.pallas import tpu_sc as plsc

B = 4; K_DIM = 128; L = 96; S = 2048; KNEI = 16; NHEADS = 4; DH = K_DIM // NHEADS; NL = 2
LL = L * L
BS = B * S
GR = B * LL
NW = 32
RPW = BS // NW
HP = DH // 2
SCALE = 1.0 / (DH ** 0.5)
EPS = 1e-5

_MESH = plsc.VectorSubcoreMesh(core_axis_name="c", subcore_axis_name="s")
_SC_PARAMS = pltpu.CompilerParams(needs_layout_passes=False,
                                  use_tc_tiling_on_sc=False)


def _wid():
    return lax.axis_index("c") * 16 + lax.axis_index("s")


def _sc_gather_rows_body(src_hbm, gidx_hbm, out_hbm, idx_v, row_v, sem):
    w = _wid()
    pltpu.sync_copy(gidx_hbm.at[w], idx_v)
    for c in range(2):
        pltpu.async_copy(src_hbm.at[idx_v.at[c]], row_v, sem).wait()
        pltpu.sync_copy(row_v, out_hbm.at[pl.ds(w * RPW + c * 128, 128)])


def _sc_gather_rows(src, gidx, n_src_rows):
    return pl.kernel(
        _sc_gather_rows_body,
        out_type=jax.ShapeDtypeStruct((BS, K_DIM), jnp.float32),
        mesh=_MESH,
        compiler_params=_SC_PARAMS,
        scratch_types=[
            pltpu.VMEM((2, 128), jnp.int32),
            pltpu.VMEM((128, K_DIM), jnp.float32),
            pltpu.SemaphoreType.DMA,
        ],
    )(src, gidx)


def _sc_attn_body(qt, kf, vf, nidx_hbm, wres_hbm, o_hbm, ores_hbm,
                  tab, idxb, scT, qob, obuf, widx_v, rbuf, sem):
    cid = lax.axis_index("c")
    sid = lax.axis_index("s")
    b = 2 * cid + sid // 8
    u = sid % 8
    h = u // 2
    half = u % 2
    qbase0 = b * S + half * 1024
    col0 = 32 * h
    iota = lax.iota(jnp.int32, 16)

    def fc(v):
        return jnp.full((16,), v, jnp.int32)

    pltpu.sync_copy(nidx_hbm.at[:, pl.ds(qbase0, 1024)], idxb)

    hp0 = 16 * h
    pltpu.sync_copy(kf.at[pl.ds(b * S, S), pl.ds(hp0, HP)], tab.at[:, pl.ds(0, HP)])

    def chunk_a(c, _):
        pltpu.sync_copy(qt.at[pl.ds(col0, DH), pl.ds(qbase0 + c * 128, 128)], qob)

        def group_a(g, _):
            srow = c * 128 + g * 16
            ni = [idxb[k, pl.ds(srow, 16)] for k in range(KNEI)]
            acc = [None] * KNEI
            for j in range(HP):
                qa = qob[2 * j, pl.ds(g * 16, 16)]
                qb = qob[2 * j + 1, pl.ds(g * 16, 16)]
                for k in range(KNEI):
                    wd = plsc.load_gather(tab, [ni[k], fc(j)])
                    ea = plsc.bitcast(lax.shift_left(wd, 16), jnp.float32)
                    eb = plsc.bitcast(wd & jnp.int32(-65536), jnp.float32)
                    t = qa * ea + qb * eb
                    acc[k] = t if acc[k] is None else acc[k] + t
            for k in range(KNEI):
                scT[k, pl.ds(srow, 16)] = acc[k] * SCALE
            return 0

        lax.fori_loop(0, 8, group_a, 0)
        return 0

    lax.fori_loop(0, 8, chunk_a, 0)

    pltpu.sync_copy(vf.at[pl.ds(b * S, S), pl.ds(hp0, HP)], tab.at[:, pl.ds(0, HP)])

    def chunk_b(c, _):
        def group_b(g, _):
            srow = c * 128 + g * 16
            sc = [scT[k, pl.ds(srow, 16)] for k in range(KNEI)]
            m = sc[0]
            for k in range(1, KNEI):
                m = jnp.maximum(m, sc[k])
            ek = [jnp.exp(sc[k] - m) for k in range(KNEI)]
            den = ek[0]
            for k in range(1, KNEI):
                den = den + ek[k]
            inv = 1.0 / den
            wk = [ek[k] * inv for k in range(KNEI)]
            ni = [idxb[k, pl.ds(srow, 16)] for k in range(KNEI)]
            orow = iota + g * 16
            for jb in range(4):
                od = [None] * 8
                for k in range(KNEI):
                    for ji in range(4):
                        j = jb * 4 + ji
                        wd = plsc.load_gather(tab, [ni[k], fc(j)])
                        ea = plsc.bitcast(lax.shift_left(wd, 16), jnp.float32)
                        eb = plsc.bitcast(wd & jnp.int32(-65536), jnp.float32)
                        t0 = wk[k] * ea
                        t1 = wk[k] * eb
                        od[2 * ji] = t0 if od[2 * ji] is None else od[2 * ji] + t0
                        od[2 * ji + 1] = t1 if od[2 * ji + 1] is None else od[2 * ji + 1] + t1
                for ji in range(4):
                    j = jb * 4 + ji
                    plsc.store_scatter(obuf, [orow, fc(2 * j)], od[2 * ji])
                    plsc.store_scatter(obuf, [orow, fc(2 * j + 1)], od[2 * ji + 1])
            return 0

        lax.fori_loop(0, 8, group_b, 0)
        pltpu.sync_copy(obuf.at[:, pl.ds(0, DH)],
                        o_hbm.at[pl.ds(qbase0 + c * 128, 128), pl.ds(col0, DH)])
        return 0

    lax.fori_loop(0, 8, chunk_b, 0)

    plsc.subcore_barrier()
    w = _wid()
    pltpu.sync_copy(wres_hbm.at[w], widx_v)
    for c in range(8):
        pltpu.async_copy(o_hbm.at[widx_v.at[c]], rbuf, sem).wait()
        pltpu.sync_copy(rbuf, ores_hbm.at[pl.ds(w * RPW + c * 32, 32)])


def _sc_attn(qf, kf, vf, nidx, wres):
    return pl.kernel(
        _sc_attn_body,
        out_type=(jax.ShapeDtypeStruct((BS, K_DIM), jnp.float32),
                  jax.ShapeDtypeStruct((BS, K_DIM), jnp.float32)),
        mesh=_MESH,
        compiler_params=_SC_PARAMS,
        scratch_types=[
            pltpu.VMEM((S, HP + 1), jnp.int32),
            pltpu.VMEM((KNEI, 1024), jnp.int32),
            pltpu.VMEM((KNEI, 1024), jnp.float32),
            pltpu.VMEM((DH, 128), jnp.float32),
            pltpu.VMEM((128, DH + 1), jnp.float32),
            pltpu.VMEM((8, 32), jnp.int32),
            pltpu.VMEM((32, K_DIM), jnp.float32),
            pltpu.SemaphoreType.DMA,
        ],
    )(qf, kf, vf, nidx, wres)


def _sc_assemble_body(ugrid, rows_f, sidx_hbm, out_hbm, big_v, idx_v, row_v, sem):
    cid = lax.axis_index("c")
    sid = lax.axis_index("s")
    copy_base = cid * (GR // 2) + sid * (GR // NW)
    for c in range(2):
        off = copy_base + c * 576
        pltpu.sync_copy(ugrid.at[pl.ds(off, 576)], big_v)
        pltpu.sync_copy(big_v, out_hbm.at[pl.ds(off, 576)])
    plsc.subcore_barrier()
    w = cid * 16 + sid
    pltpu.sync_copy(sidx_hbm.at[w], idx_v)
    for c in range(2):
        pltpu.sync_copy(rows_f.at[pl.ds(w * RPW + c * 128, 128)], row_v)
        pltpu.async_copy(row_v, out_hbm.at[idx_v.at[c]], sem).wait()


def _sc_assemble(ugrid, rows_f, sidx):
    return pl.kernel(
        _sc_assemble_body,
        out_type=jax.ShapeDtypeStruct((GR, K_DIM), jnp.float32),
        mesh=_MESH,
        compiler_params=_SC_PARAMS,
        scratch_types=[
            pltpu.VMEM((576, K_DIM), jnp.float32),
            pltpu.VMEM((2, 128), jnp.int32),
            pltpu.VMEM((128, K_DIM), jnp.float32),
            pltpu.SemaphoreType.DMA,
        ],
    )(ugrid, rows_f, sidx)


def _ln_rows(x, s_ref, b_ref):
    mu = jnp.mean(x, axis=-1, keepdims=True)
    var = jnp.mean((x - mu) ** 2, axis=-1, keepdims=True)
    return (x - mu) * lax.rsqrt(var + EPS) * s_ref[...] + b_ref[...]


def _matT(x, w_ref):
    return lax.dot_general(x, w_ref[...], (((1,), (1,)), ((), ())),
                           preferred_element_type=jnp.float32)


def _tc_qkv1_body(h_ref, lis, lib, wq, wk, wv, q_ref, k_ref, v_ref):
    hn = _ln_rows(h_ref[...], lis, lib)
    q_ref[...] = _matT(hn, wq).T
    k_ref[...] = _matT(hn, wk)
    v_ref[...] = _matT(hn, wv)


def _tc_upd_qkv_body(prev_ref, ores_ref, wo, lcs, lcb, lis, lib, wq, wk, wv,
                     rows_ref, q_ref, k_ref, v_ref):
    r2 = _ln_rows(prev_ref[...] + _matT(ores_ref[...], wo), lcs, lcb)
    rows_ref[...] = r2
    hn = _ln_rows(r2, lis, lib)
    q_ref[...] = _matT(hn, wq).T
    k_ref[...] = _matT(hn, wk)
    v_ref[...] = _matT(hn, wv)


def _tc_upd_body(prev_ref, ores_ref, wo, lcs, lcb, rows_ref):
    rows_ref[...] = _ln_rows(prev_ref[...] + _matT(ores_ref[...], wo), lcs, lcb)


def _tc_gridln_body(x_ref, s1, b1, s2, b2, o_ref):
    o_ref[...] = _ln_rows(_ln_rows(x_ref[...], s1, b1), s2, b2)


def _rows_spec(blk):
    return pl.BlockSpec((blk, K_DIM), lambda i: (i, 0))


def _full_spec(shape):
    return pl.BlockSpec(shape, lambda i: tuple(0 for _ in shape))


_VEC = _full_spec((1, K_DIM))
_WMAT = _full_spec((K_DIM, K_DIM))
_ROWS_T = jax.ShapeDtypeStruct((BS, K_DIM), jnp.float32)
_QT_T = jax.ShapeDtypeStruct((K_DIM, BS), jnp.float32)


def _qt_spec():
    return pl.BlockSpec((K_DIM, 1024), lambda i: (0, i))


def _tc_qkv1(h, lis, lib, wq, wk, wv):
    return pl.pallas_call(
        _tc_qkv1_body, grid=(8,),
        in_specs=[_rows_spec(1024), _VEC, _VEC, _WMAT, _WMAT, _WMAT],
        out_specs=[_qt_spec(), _rows_spec(1024), _rows_spec(1024)],
        out_shape=[_QT_T, _ROWS_T, _ROWS_T],
    )(h, lis, lib, wq, wk, wv)


def _tc_upd_qkv(prev, ores, wo, lcs, lcb, lis, lib, wq, wk, wv):
    return pl.pallas_call(
        _tc_upd_qkv_body, grid=(8,),
        in_specs=[_rows_spec(1024), _rows_spec(1024), _WMAT, _VEC, _VEC,
                  _VEC, _VEC, _WMAT, _WMAT, _WMAT],
        out_specs=[_rows_spec(1024), _qt_spec(), _rows_spec(1024), _rows_spec(1024)],
        out_shape=[_ROWS_T, _QT_T, _ROWS_T, _ROWS_T],
    )(prev, ores, wo, lcs, lcb, lis, lib, wq, wk, wv)


def _tc_upd(prev, ores, wo, lcs, lcb):
    return pl.pallas_call(
        _tc_upd_body, grid=(8,),
        in_specs=[_rows_spec(1024), _rows_spec(1024), _WMAT, _VEC, _VEC],
        out_specs=_rows_spec(1024),
        out_shape=_ROWS_T,
    )(prev, ores, wo, lcs, lcb)


def _tc_gridln(gf, s1, b1, s2, b2):
    return pl.pallas_call(
        _tc_gridln_body, grid=(9,),
        in_specs=[_rows_spec(4096), _VEC, _VEC, _VEC, _VEC],
        out_specs=_rows_spec(4096),
        out_shape=jax.ShapeDtypeStruct((GR, K_DIM), jnp.float32),
    )(gf, s1, b1, s2, b2)


def kernel(grid_scores, N_idx, N_mask, id2lr_pad, S_row_mask, Wq, Wk, Wv, Wo,
           ln_in_scale, ln_in_bias, ln_ch_scale, ln_ch_bias):
    f32 = jnp.float32
    lin = jnp.clip(id2lr_pad[..., 0] * L + id2lr_pad[..., 1], 0, LL - 1).astype(jnp.int32)
    barange = jnp.broadcast_to(jnp.arange(S, dtype=jnp.int32)[None], (B, S))
    wcell = jnp.zeros((B, LL), jnp.int32).at[jnp.arange(B)[:, None], lin].set(barange)
    w_s = jnp.take_along_axis(wcell, lin, axis=1)
    n32 = jnp.clip(N_idx, 0, S - 1).astype(jnp.int32).reshape(BS, KNEI).T
    gidx = (jnp.arange(B, dtype=jnp.int32)[:, None] * LL + lin).reshape(NW, 2, 128)
    wres = (jnp.arange(B, dtype=jnp.int32)[:, None] * S + w_s).reshape(NW, 8, 32)

    gf = grid_scores.reshape(B, K_DIM, LL).transpose(0, 2, 1).reshape(GR, K_DIM)
    lv = [a.reshape(NL, 1, K_DIM).astype(f32)
          for a in (ln_in_scale, ln_in_bias, ln_ch_scale, ln_ch_bias)]
    lis, lib, lcs, lcb = lv

    rows = _sc_gather_rows(gf, gidx, GR)

    def packpairs(x):
        return jax.lax.bitcast_convert_type(
            x.astype(jnp.bfloat16).reshape(BS, K_DIM // 2, 2), jnp.int32)

    q1, k1, v1 = _tc_qkv1(rows, lis[0], lib[0], Wq[0], Wk[0], Wv[0])
    _, ores1 = _sc_attn(q1, packpairs(k1), packpairs(v1), n32, wres)
    rows2, q2, k2, v2 = _tc_upd_qkv(rows, ores1, Wo[0], lcs[0], lcb[0],
                                    lis[1], lib[1], Wq[1], Wk[1], Wv[1])
    _, ores2 = _sc_attn(q2, packpairs(k2), packpairs(v2), n32, wres)
    rows_f = _tc_upd(rows2, ores2, Wo[1], lcs[1], lcb[1])

    ugrid = _tc_gridln(gf, lcs[0], lcb[0], lcs[1], lcb[1])
    out = _sc_assemble(ugrid, rows_f, gidx)

    return out.reshape(B, LL, K_DIM).transpose(0, 2, 1).reshape(B, K_DIM, L, L)


# speedup vs baseline: 2.3847x; 1.0193x over previous
import functools
import jax
import jax.numpy as jnp
from jax import lax
from jax.experimental import pallas as pl
from jax.experimental.pallas import tpu as pltpu
from jax.experimental.pallas import tpu_sc as plsc

B = 4; K_DIM = 128; L = 96; S = 2048; KNEI = 16; NHEADS = 4; DH = K_DIM // NHEADS; NL = 2
LL = L * L
BS = B * S
GR = B * LL
NW = 32
RPW = BS // NW
HP = DH // 2
SCALE = 1.0 / (DH ** 0.5)
EPS = 1e-5

_MESH = plsc.VectorSubcoreMesh(core_axis_name="c", subcore_axis_name="s")
_SC_PARAMS = pltpu.CompilerParams(needs_layout_passes=False,
                                  use_tc_tiling_on_sc=False)


def _wid():
    return lax.axis_index("c") * 16 + lax.axis_index("s")


def _sc_gather_rows_body(src_hbm, gidx_hbm, out_hbm, idx_v, row_v, sem):
    w = _wid()
    pltpu.sync_copy(gidx_hbm.at[w], idx_v)
    for c in range(2):
        pltpu.async_copy(src_hbm.at[idx_v.at[c]], row_v, sem).wait()
        pltpu.sync_copy(row_v, out_hbm.at[pl.ds(w * RPW + c * 128, 128)])


def _sc_gather_rows(src, gidx, n_src_rows):
    return pl.kernel(
        _sc_gather_rows_body,
        out_type=jax.ShapeDtypeStruct((BS, K_DIM), jnp.float32),
        mesh=_MESH,
        compiler_params=_SC_PARAMS,
        scratch_types=[
            pltpu.VMEM((2, 128), jnp.int32),
            pltpu.VMEM((128, K_DIM), jnp.float32),
            pltpu.SemaphoreType.DMA,
        ],
    )(src, gidx)


def _sc_attn_body(qt, kf, vf, nidx_hbm, wres_hbm, o_hbm, ores_hbm,
                  tab, idxb, scT, qob, obuf, widx_v, rbuf,
                  qs0, qs1, os0, os1, rs0, rs1):
    cid = lax.axis_index("c")
    sid = lax.axis_index("s")
    b = 2 * cid + sid // 8
    u = sid % 8
    h = u // 2
    half = u % 2
    qbase0 = b * S + half * 1024
    col0 = 32 * h
    hp0 = 16 * h
    iota = lax.iota(jnp.int32, 16)

    def fc(v):
        return jnp.full((16,), v, jnp.int32)

    qsems = (qs0, qs1)
    osems = (os0, os1)
    rsems = (rs0, rs1)

    def qcopy(c, slot):
        return pltpu.make_async_copy(
            qt.at[pl.ds(col0, DH), pl.ds(qbase0 + c * 128, 128)],
            qob.at[slot], qsems[slot])

    def ocopy(c, slot):
        return pltpu.make_async_copy(
            obuf.at[slot, :, pl.ds(0, DH)],
            o_hbm.at[pl.ds(qbase0 + c * 128, 128), pl.ds(col0, DH)],
            osems[slot])

    pltpu.sync_copy(nidx_hbm.at[:, pl.ds(qbase0, 1024)], idxb)

    pltpu.sync_copy(kf.at[pl.ds(b * S, S), pl.ds(hp0, HP)], tab.at[:, pl.ds(0, HP)])
    qcopy(0, 0).start()

    def groups_a(c, slot):
        def group_a(g, _):
            srow = c * 128 + g * 16
            ni = [idxb[k, pl.ds(srow, 16)] for k in range(KNEI)]
            acc = [None] * KNEI
            for j in range(HP):
                qa = qob[slot, 2 * j, pl.ds(g * 16, 16)]
                qb = qob[slot, 2 * j + 1, pl.ds(g * 16, 16)]
                for k in range(KNEI):
                    wd = plsc.load_gather(tab, [ni[k], fc(j)])
                    ea = plsc.bitcast(lax.shift_left(wd, 16), jnp.float32)
                    eb = plsc.bitcast(wd & jnp.int32(-65536), jnp.float32)
                    t = qa * ea + qb * eb
                    acc[k] = t if acc[k] is None else acc[k] + t
            for k in range(KNEI):
                scT[k, pl.ds(srow, 16)] = acc[k] * SCALE
            return 0

        lax.fori_loop(0, 8, group_a, 0)

    def bigchunk_a(cc, _):
        c0 = 2 * cc
        qcopy(c0 + 1, 1).start()
        qcopy(c0, 0).wait()
        groups_a(c0, 0)

        @pl.when(cc < 3)
        def _():
            qcopy(c0 + 2, 0).start()

        qcopy(c0 + 1, 1).wait()
        groups_a(c0 + 1, 1)
        return 0

    lax.fori_loop(0, 4, bigchunk_a, 0)

    pltpu.sync_copy(vf.at[pl.ds(b * S, S), pl.ds(hp0, HP)], tab.at[:, pl.ds(0, HP)])

    def groups_b(c, slot):
        def group_b(g, _):
            srow = c * 128 + g * 16
            sc = [scT[k, pl.ds(srow, 16)] for k in range(KNEI)]
            m = sc[0]
            for k in range(1, KNEI):
                m = jnp.maximum(m, sc[k])
            ek = [jnp.exp(sc[k] - m) for k in range(KNEI)]
            den = ek[0]
            for k in range(1, KNEI):
                den = den + ek[k]
            inv = 1.0 / den
            wk = [ek[k] * inv for k in range(KNEI)]
            ni = [idxb[k, pl.ds(srow, 16)] for k in range(KNEI)]
            orow = iota + g * 16
            for jb in range(4):
                od = [None] * 8
                for k in range(KNEI):
                    for ji in range(4):
                        j = jb * 4 + ji
                        wd = plsc.load_gather(tab, [ni[k], fc(j)])
                        ea = plsc.bitcast(lax.shift_left(wd, 16), jnp.float32)
                        eb = plsc.bitcast(wd & jnp.int32(-65536), jnp.float32)
                        t0 = wk[k] * ea
                        t1 = wk[k] * eb
                        od[2 * ji] = t0 if od[2 * ji] is None else od[2 * ji] + t0
                        od[2 * ji + 1] = t1 if od[2 * ji + 1] is None else od[2 * ji + 1] + t1
                for ji in range(4):
                    j = jb * 4 + ji
                    plsc.store_scatter(obuf.at[slot], [orow, fc(2 * j)], od[2 * ji])
                    plsc.store_scatter(obuf.at[slot], [orow, fc(2 * j + 1)], od[2 * ji + 1])
            return 0

        lax.fori_loop(0, 8, group_b, 0)

    def bigchunk_b(cc, _):
        c0 = 2 * cc

        @pl.when(cc > 0)
        def _():
            ocopy(c0, 0).wait()

        groups_b(c0, 0)
        ocopy(c0, 0).start()

        @pl.when(cc > 0)
        def _():
            ocopy(c0 + 1, 1).wait()

        groups_b(c0 + 1, 1)
        ocopy(c0 + 1, 1).start()
        return 0

    lax.fori_loop(0, 4, bigchunk_b, 0)
    ocopy(6, 0).wait()
    ocopy(7, 1).wait()

    plsc.subcore_barrier()
    w = _wid()
    pltpu.sync_copy(wres_hbm.at[w], widx_v)

    def rcopy(c, slot):
        return pltpu.make_async_copy(o_hbm.at[widx_v.at[c]], rbuf.at[slot],
                                     rsems[slot])

    rcopy(0, 0).start()
    rcopy(1, 1).start()
    for c in range(4):
        s = c % 2
        rcopy(c, s).wait()
        pltpu.sync_copy(rbuf.at[s], ores_hbm.at[pl.ds(w * RPW + c * 64, 64)])
        if c + 2 < 4:
            rcopy(c + 2, s).start()


def _sc_attn(qf, kf, vf, nidx, wres):
    return pl.kernel(
        _sc_attn_body,
        out_type=(jax.ShapeDtypeStruct((BS, K_DIM), jnp.float32),
                  jax.ShapeDtypeStruct((BS, K_DIM), jnp.float32)),
        mesh=_MESH,
        compiler_params=_SC_PARAMS,
        scratch_types=[
            pltpu.VMEM((S, HP + 1), jnp.int32),
            pltpu.VMEM((KNEI, 1024), jnp.int32),
            pltpu.VMEM((KNEI, 1024), jnp.float32),
            pltpu.VMEM((2, DH, 128), jnp.float32),
            pltpu.VMEM((2, 128, DH + 1), jnp.float32),
            pltpu.VMEM((4, 64), jnp.int32),
            pltpu.VMEM((2, 64, K_DIM), jnp.float32),
            pltpu.SemaphoreType.DMA,
            pltpu.SemaphoreType.DMA,
            pltpu.SemaphoreType.DMA,
            pltpu.SemaphoreType.DMA,
            pltpu.SemaphoreType.DMA,
            pltpu.SemaphoreType.DMA,
        ],
    )(qf, kf, vf, nidx, wres)


def _sc_assemble_body(ugrid, rows_f, sidx_hbm, out_hbm, big_v, idx_v, row_v, sem):
    cid = lax.axis_index("c")
    sid = lax.axis_index("s")
    copy_base = cid * (GR // 2) + sid * (GR // NW)
    for c in range(2):
        off = copy_base + c * 576
        pltpu.sync_copy(ugrid.at[pl.ds(off, 576)], big_v)
        pltpu.sync_copy(big_v, out_hbm.at[pl.ds(off, 576)])
    plsc.subcore_barrier()
    w = cid * 16 + sid
    pltpu.sync_copy(sidx_hbm.at[w], idx_v)
    for c in range(2):
        pltpu.sync_copy(rows_f.at[pl.ds(w * RPW + c * 128, 128)], row_v)
        pltpu.async_copy(row_v, out_hbm.at[idx_v.at[c]], sem).wait()


def _sc_assemble(ugrid, rows_f, sidx):
    return pl.kernel(
        _sc_assemble_body,
        out_type=jax.ShapeDtypeStruct((GR, K_DIM), jnp.float32),
        mesh=_MESH,
        compiler_params=_SC_PARAMS,
        scratch_types=[
            pltpu.VMEM((576, K_DIM), jnp.float32),
            pltpu.VMEM((2, 128), jnp.int32),
            pltpu.VMEM((128, K_DIM), jnp.float32),
            pltpu.SemaphoreType.DMA,
        ],
    )(ugrid, rows_f, sidx)


def _ln_rows(x, s_ref, b_ref):
    mu = jnp.mean(x, axis=-1, keepdims=True)
    var = jnp.mean((x - mu) ** 2, axis=-1, keepdims=True)
    return (x - mu) * lax.rsqrt(var + EPS) * s_ref[...] + b_ref[...]


def _matT(x, w_ref):
    return lax.dot_general(x, w_ref[...], (((1,), (1,)), ((), ())),
                           preferred_element_type=jnp.float32)


def _tc_qkv1_body(h_ref, lis, lib, wq, wk, wv, q_ref, k_ref, v_ref):
    hn = _ln_rows(h_ref[...], lis, lib)
    q_ref[...] = _matT(hn, wq).T
    k_ref[...] = _matT(hn, wk)
    v_ref[...] = _matT(hn, wv)


def _tc_upd_qkv_body(prev_ref, ores_ref, wo, lcs, lcb, lis, lib, wq, wk, wv,
                     rows_ref, q_ref, k_ref, v_ref):
    r2 = _ln_rows(prev_ref[...] + _matT(ores_ref[...], wo), lcs, lcb)
    rows_ref[...] = r2
    hn = _ln_rows(r2, lis, lib)
    q_ref[...] = _matT(hn, wq).T
    k_ref[...] = _matT(hn, wk)
    v_ref[...] = _matT(hn, wv)


def _tc_upd_body(prev_ref, ores_ref, wo, lcs, lcb, rows_ref):
    rows_ref[...] = _ln_rows(prev_ref[...] + _matT(ores_ref[...], wo), lcs, lcb)


def _tc_gridln_body(x_ref, s1, b1, s2, b2, o_ref):
    o_ref[...] = _ln_rows(_ln_rows(x_ref[...], s1, b1), s2, b2)


def _rows_spec(blk):
    return pl.BlockSpec((blk, K_DIM), lambda i: (i, 0))


def _full_spec(shape):
    return pl.BlockSpec(shape, lambda i: tuple(0 for _ in shape))


_VEC = _full_spec((1, K_DIM))
_WMAT = _full_spec((K_DIM, K_DIM))
_ROWS_T = jax.ShapeDtypeStruct((BS, K_DIM), jnp.float32)
_QT_T = jax.ShapeDtypeStruct((K_DIM, BS), jnp.float32)


def _qt_spec():
    return pl.BlockSpec((K_DIM, 1024), lambda i: (0, i))


_PK_T = jax.ShapeDtypeStruct((BS, K_DIM // 2), jnp.int32)


def _pk_spec():
    return pl.BlockSpec((1024, K_DIM // 2), lambda i: (i, 0))


def _tc_qkv1(h, lis, lib, wq, wk, wv):
    return pl.pallas_call(
        _tc_qkv1_body, grid=(8,),
        in_specs=[_rows_spec(1024), _VEC, _VEC, _WMAT, _WMAT, _WMAT],
        out_specs=[_qt_spec(), _rows_spec(1024), _rows_spec(1024)],
        out_shape=[_QT_T, _ROWS_T, _ROWS_T],
    )(h, lis, lib, wq, wk, wv)


def _tc_upd_qkv(prev, ores, wo, lcs, lcb, lis, lib, wq, wk, wv):
    return pl.pallas_call(
        _tc_upd_qkv_body, grid=(8,),
        in_specs=[_rows_spec(1024), _rows_spec(1024), _WMAT, _VEC, _VEC,
                  _VEC, _VEC, _WMAT, _WMAT, _WMAT],
        out_specs=[_rows_spec(1024), _qt_spec(), _rows_spec(1024), _rows_spec(1024)],
        out_shape=[_ROWS_T, _QT_T, _ROWS_T, _ROWS_T],
    )(prev, ores, wo, lcs, lcb, lis, lib, wq, wk, wv)


def _tc_upd(prev, ores, wo, lcs, lcb):
    return pl.pallas_call(
        _tc_upd_body, grid=(8,),
        in_specs=[_rows_spec(1024), _rows_spec(1024), _WMAT, _VEC, _VEC],
        out_specs=_rows_spec(1024),
        out_shape=_ROWS_T,
    )(prev, ores, wo, lcs, lcb)


def _tc_gridln(gf, s1, b1, s2, b2):
    return pl.pallas_call(
        _tc_gridln_body, grid=(9,),
        in_specs=[_rows_spec(4096), _VEC, _VEC, _VEC, _VEC],
        out_specs=_rows_spec(4096),
        out_shape=jax.ShapeDtypeStruct((GR, K_DIM), jnp.float32),
    )(gf, s1, b1, s2, b2)


def kernel(grid_scores, N_idx, N_mask, id2lr_pad, S_row_mask, Wq, Wk, Wv, Wo,
           ln_in_scale, ln_in_bias, ln_ch_scale, ln_ch_bias):
    f32 = jnp.float32
    lin = jnp.clip(id2lr_pad[..., 0] * L + id2lr_pad[..., 1], 0, LL - 1).astype(jnp.int32)
    barange = jnp.broadcast_to(jnp.arange(S, dtype=jnp.int32)[None], (B, S))
    wcell = jnp.zeros((B, LL), jnp.int32).at[jnp.arange(B)[:, None], lin].set(barange)
    w_s = jnp.take_along_axis(wcell, lin, axis=1)
    n32 = jnp.clip(N_idx, 0, S - 1).astype(jnp.int32).reshape(BS, KNEI).T
    gidx = (jnp.arange(B, dtype=jnp.int32)[:, None] * LL + lin).reshape(NW, 2, 128)
    wres = (jnp.arange(B, dtype=jnp.int32)[:, None] * S + w_s).reshape(NW, 4, 64)

    gf = grid_scores.reshape(B, K_DIM, LL).transpose(0, 2, 1).reshape(GR, K_DIM)
    lv = [a.reshape(NL, 1, K_DIM).astype(f32)
          for a in (ln_in_scale, ln_in_bias, ln_ch_scale, ln_ch_bias)]
    lis, lib, lcs, lcb = lv

    rows = _sc_gather_rows(gf, gidx, GR)

    def packpairs(x):
        return jax.lax.bitcast_convert_type(
            x.astype(jnp.bfloat16).reshape(BS, K_DIM // 2, 2), jnp.int32)

    q1, k1, v1 = _tc_qkv1(rows, lis[0], lib[0], Wq[0], Wk[0], Wv[0])
    _, ores1 = _sc_attn(q1, packpairs(k1), packpairs(v1), n32, wres)
    rows2, q2, k2, v2 = _tc_upd_qkv(rows, ores1, Wo[0], lcs[0], lcb[0],
                                    lis[1], lib[1], Wq[1], Wk[1], Wv[1])
    _, ores2 = _sc_attn(q2, packpairs(k2), packpairs(v2), n32, wres)
    rows_f = _tc_upd(rows2, ores2, Wo[1], lcs[1], lcb[1])

    ugrid = _tc_gridln(gf, lcs[0], lcb[0], lcs[1], lcb[1])
    out = _sc_assemble(ugrid, rows_f, gidx)

    return out.reshape(B, LL, K_DIM).transpose(0, 2, 1).reshape(B, K_DIM, L, L)


# speedup vs baseline: 2.5072x; 1.0514x over previous
import functools
import jax
import jax.numpy as jnp
from jax import lax
from jax.experimental import pallas as pl
from jax.experimental.pallas import tpu as pltpu
from jax.experimental.pallas import tpu_sc as plsc

B = 4; K_DIM = 128; L = 96; S = 2048; KNEI = 16; NHEADS = 4; DH = K_DIM // NHEADS; NL = 2
LL = L * L
BS = B * S
GR = B * LL
NW = 32
RPW = BS // NW
HP = DH // 2
SCALE = 1.0 / (DH ** 0.5)
EPS = 1e-5

_MESH = plsc.VectorSubcoreMesh(core_axis_name="c", subcore_axis_name="s")
_SC_PARAMS = pltpu.CompilerParams(needs_layout_passes=False,
                                  use_tc_tiling_on_sc=False)


def _wid():
    return lax.axis_index("c") * 16 + lax.axis_index("s")


def _sc_gather_rows_body(src_hbm, gidx_hbm, out_hbm, idx_v, row_v, sem):
    w = _wid()
    pltpu.sync_copy(gidx_hbm.at[w], idx_v)
    for c in range(2):
        pltpu.async_copy(src_hbm.at[idx_v.at[c]], row_v, sem).wait()
        pltpu.sync_copy(row_v, out_hbm.at[pl.ds(w * RPW + c * 128, 128)])


def _sc_gather_rows(src, gidx, n_src_rows):
    return pl.kernel(
        _sc_gather_rows_body,
        out_type=jax.ShapeDtypeStruct((BS, K_DIM), jnp.float32),
        mesh=_MESH,
        compiler_params=_SC_PARAMS,
        scratch_types=[
            pltpu.VMEM((2, 128), jnp.int32),
            pltpu.VMEM((128, K_DIM), jnp.float32),
            pltpu.SemaphoreType.DMA,
        ],
    )(src, gidx)


def _sc_attn_body(qt, kf, vf, nidx_hbm, wres_hbm, o_hbm, ores_hbm,
                  tab, idxb, scT, qob, obuf, widx_v, rbuf,
                  qs0, qs1, os0, os1, rs0, rs1):
    cid = lax.axis_index("c")
    sid = lax.axis_index("s")
    b = 2 * cid + sid // 8
    u = sid % 8
    h = u // 2
    half = u % 2
    qbase0 = b * S + half * 1024
    col0 = 32 * h
    hp0 = 16 * h
    iota = lax.iota(jnp.int32, 16)

    def fc(v):
        return jnp.full((16,), v, jnp.int32)

    qsems = (qs0, qs1)
    osems = (os0, os1)
    rsems = (rs0, rs1)

    def qcopy(c, slot):
        return pltpu.make_async_copy(
            qt.at[pl.ds(col0, DH), pl.ds(qbase0 + c * 128, 128)],
            qob.at[slot], qsems[slot])

    def ocopy(c, slot):
        return pltpu.make_async_copy(
            obuf.at[slot, :, pl.ds(0, DH)],
            o_hbm.at[pl.ds(qbase0 + c * 128, 128), pl.ds(col0, DH)],
            osems[slot])

    pltpu.sync_copy(nidx_hbm.at[:, pl.ds(qbase0, 1024)], idxb)

    pltpu.sync_copy(kf.at[pl.ds(b * S, S), pl.ds(hp0, HP)], tab.at[:, pl.ds(0, HP)])
    qcopy(0, 0).start()

    def groups_a(c, slot):
        def group_a(g, _):
            srow = c * 128 + g * 16
            ni = [idxb[k, pl.ds(srow, 16)] for k in range(KNEI)]
            acc = [None] * KNEI
            for j in range(HP):
                qa = qob[slot, 2 * j, pl.ds(g * 16, 16)]
                qb = qob[slot, 2 * j + 1, pl.ds(g * 16, 16)]
                for k in range(KNEI):
                    wd = plsc.load_gather(tab, [ni[k], fc(j)])
                    ea = plsc.bitcast(lax.shift_left(wd, 16), jnp.float32)
                    eb = plsc.bitcast(wd, jnp.float32)
                    t = qa * ea + qb * eb
                    acc[k] = t if acc[k] is None else acc[k] + t
            for k in range(KNEI):
                scT[k, pl.ds(srow, 16)] = acc[k]
            return 0

        lax.fori_loop(0, 8, group_a, 0)

    def bigchunk_a(cc, _):
        c0 = 2 * cc
        qcopy(c0 + 1, 1).start()
        qcopy(c0, 0).wait()
        groups_a(c0, 0)

        @pl.when(cc < 3)
        def _():
            qcopy(c0 + 2, 0).start()

        qcopy(c0 + 1, 1).wait()
        groups_a(c0 + 1, 1)
        return 0

    lax.fori_loop(0, 4, bigchunk_a, 0)

    pltpu.sync_copy(vf.at[pl.ds(b * S, S), pl.ds(hp0, HP)], tab.at[:, pl.ds(0, HP)])

    def groups_b(c, slot):
        def group_b(g, _):
            srow = c * 128 + g * 16
            sc = [scT[k, pl.ds(srow, 16)] for k in range(KNEI)]
            m = sc[0]
            for k in range(1, KNEI):
                m = jnp.maximum(m, sc[k])
            ek = [jnp.exp(sc[k] - m) for k in range(KNEI)]
            den = ek[0]
            for k in range(1, KNEI):
                den = den + ek[k]
            inv = 1.0 / den
            wk = [ek[k] * inv for k in range(KNEI)]
            ni = [idxb[k, pl.ds(srow, 16)] for k in range(KNEI)]
            orow = iota + g * 16
            for jb in range(4):
                od = [None] * 8
                for k in range(KNEI):
                    for ji in range(4):
                        j = jb * 4 + ji
                        wd = plsc.load_gather(tab, [ni[k], fc(j)])
                        ea = plsc.bitcast(lax.shift_left(wd, 16), jnp.float32)
                        eb = plsc.bitcast(wd, jnp.float32)
                        t0 = wk[k] * ea
                        t1 = wk[k] * eb
                        od[2 * ji] = t0 if od[2 * ji] is None else od[2 * ji] + t0
                        od[2 * ji + 1] = t1 if od[2 * ji + 1] is None else od[2 * ji + 1] + t1
                for ji in range(4):
                    j = jb * 4 + ji
                    plsc.store_scatter(obuf.at[slot], [orow, fc(2 * j)], od[2 * ji])
                    plsc.store_scatter(obuf.at[slot], [orow, fc(2 * j + 1)], od[2 * ji + 1])
            return 0

        lax.fori_loop(0, 8, group_b, 0)

    def bigchunk_b(cc, _):
        c0 = 2 * cc

        @pl.when(cc > 0)
        def _():
            ocopy(c0, 0).wait()

        groups_b(c0, 0)
        ocopy(c0, 0).start()

        @pl.when(cc > 0)
        def _():
            ocopy(c0 + 1, 1).wait()

        groups_b(c0 + 1, 1)
        ocopy(c0 + 1, 1).start()
        return 0

    lax.fori_loop(0, 4, bigchunk_b, 0)
    ocopy(6, 0).wait()
    ocopy(7, 1).wait()

    plsc.subcore_barrier()
    w = _wid()
    pltpu.sync_copy(wres_hbm.at[w], widx_v)

    def rcopy(c, slot):
        return pltpu.make_async_copy(o_hbm.at[widx_v.at[c]], rbuf.at[slot],
                                     rsems[slot])

    rcopy(0, 0).start()
    rcopy(1, 1).start()
    for c in range(4):
        s = c % 2
        rcopy(c, s).wait()
        pltpu.sync_copy(rbuf.at[s], ores_hbm.at[pl.ds(w * RPW + c * 64, 64)])
        if c + 2 < 4:
            rcopy(c + 2, s).start()


def _sc_attn(qf, kf, vf, nidx, wres):
    return pl.kernel(
        _sc_attn_body,
        out_type=(jax.ShapeDtypeStruct((BS, K_DIM), jnp.float32),
                  jax.ShapeDtypeStruct((BS, K_DIM), jnp.float32)),
        mesh=_MESH,
        compiler_params=_SC_PARAMS,
        scratch_types=[
            pltpu.VMEM((S, HP + 1), jnp.int32),
            pltpu.VMEM((KNEI, 1024), jnp.int32),
            pltpu.VMEM((KNEI, 1024), jnp.float32),
            pltpu.VMEM((2, DH, 128), jnp.float32),
            pltpu.VMEM((2, 128, DH + 1), jnp.float32),
            pltpu.VMEM((4, 64), jnp.int32),
            pltpu.VMEM((2, 64, K_DIM), jnp.float32),
            pltpu.SemaphoreType.DMA,
            pltpu.SemaphoreType.DMA,
            pltpu.SemaphoreType.DMA,
            pltpu.SemaphoreType.DMA,
            pltpu.SemaphoreType.DMA,
            pltpu.SemaphoreType.DMA,
        ],
    )(qf, kf, vf, nidx, wres)


def _sc_assemble_body(ugrid, rows_f, sidx_hbm, out_hbm, big_v, idx_v, row_v, sem):
    cid = lax.axis_index("c")
    sid = lax.axis_index("s")
    copy_base = cid * (GR // 2) + sid * (GR // NW)
    for c in range(2):
        off = copy_base + c * 576
        pltpu.sync_copy(ugrid.at[pl.ds(off, 576)], big_v)
        pltpu.sync_copy(big_v, out_hbm.at[pl.ds(off, 576)])
    plsc.subcore_barrier()
    w = cid * 16 + sid
    pltpu.sync_copy(sidx_hbm.at[w], idx_v)
    for c in range(2):
        pltpu.sync_copy(rows_f.at[pl.ds(w * RPW + c * 128, 128)], row_v)
        pltpu.async_copy(row_v, out_hbm.at[idx_v.at[c]], sem).wait()


def _sc_assemble(ugrid, rows_f, sidx):
    return pl.kernel(
        _sc_assemble_body,
        out_type=jax.ShapeDtypeStruct((GR, K_DIM), jnp.float32),
        mesh=_MESH,
        compiler_params=_SC_PARAMS,
        scratch_types=[
            pltpu.VMEM((576, K_DIM), jnp.float32),
            pltpu.VMEM((2, 128), jnp.int32),
            pltpu.VMEM((128, K_DIM), jnp.float32),
            pltpu.SemaphoreType.DMA,
        ],
    )(ugrid, rows_f, sidx)


def _ln_rows(x, s_ref, b_ref):
    mu = jnp.mean(x, axis=-1, keepdims=True)
    var = jnp.mean((x - mu) ** 2, axis=-1, keepdims=True)
    return (x - mu) * lax.rsqrt(var + EPS) * s_ref[...] + b_ref[...]


def _matT(x, w_ref):
    return lax.dot_general(x, w_ref[...], (((1,), (1,)), ((), ())),
                           preferred_element_type=jnp.float32)


def _tc_qkv1_body(h_ref, lis, lib, wq, wk, wv, q_ref, k_ref, v_ref):
    hn = _ln_rows(h_ref[...], lis, lib)
    q_ref[...] = (_matT(hn, wq) * SCALE).T
    k_ref[...] = _matT(hn, wk)
    v_ref[...] = _matT(hn, wv)


def _tc_upd_qkv_body(prev_ref, ores_ref, wo, lcs, lcb, lis, lib, wq, wk, wv,
                     rows_ref, q_ref, k_ref, v_ref):
    r2 = _ln_rows(prev_ref[...] + _matT(ores_ref[...], wo), lcs, lcb)
    rows_ref[...] = r2
    hn = _ln_rows(r2, lis, lib)
    q_ref[...] = (_matT(hn, wq) * SCALE).T
    k_ref[...] = _matT(hn, wk)
    v_ref[...] = _matT(hn, wv)


def _tc_upd_body(prev_ref, ores_ref, wo, lcs, lcb, rows_ref):
    rows_ref[...] = _ln_rows(prev_ref[...] + _matT(ores_ref[...], wo), lcs, lcb)


def _tc_gridln_body(x_ref, s1, b1, s2, b2, o_ref):
    o_ref[...] = _ln_rows(_ln_rows(x_ref[...], s1, b1), s2, b2)


def _rows_spec(blk):
    return pl.BlockSpec((blk, K_DIM), lambda i: (i, 0))


def _full_spec(shape):
    return pl.BlockSpec(shape, lambda i: tuple(0 for _ in shape))


_VEC = _full_spec((1, K_DIM))
_WMAT = _full_spec((K_DIM, K_DIM))
_ROWS_T = jax.ShapeDtypeStruct((BS, K_DIM), jnp.float32)
_QT_T = jax.ShapeDtypeStruct((K_DIM, BS), jnp.float32)


def _qt_spec():
    return pl.BlockSpec((K_DIM, 1024), lambda i: (0, i))


_PK_T = jax.ShapeDtypeStruct((BS, K_DIM // 2), jnp.int32)


def _pk_spec():
    return pl.BlockSpec((1024, K_DIM // 2), lambda i: (i, 0))


def _tc_qkv1(h, lis, lib, wq, wk, wv):
    return pl.pallas_call(
        _tc_qkv1_body, grid=(8,),
        in_specs=[_rows_spec(1024), _VEC, _VEC, _WMAT, _WMAT, _WMAT],
        out_specs=[_qt_spec(), _rows_spec(1024), _rows_spec(1024)],
        out_shape=[_QT_T, _ROWS_T, _ROWS_T],
    )(h, lis, lib, wq, wk, wv)


def _tc_upd_qkv(prev, ores, wo, lcs, lcb, lis, lib, wq, wk, wv):
    return pl.pallas_call(
        _tc_upd_qkv_body, grid=(8,),
        in_specs=[_rows_spec(1024), _rows_spec(1024), _WMAT, _VEC, _VEC,
                  _VEC, _VEC, _WMAT, _WMAT, _WMAT],
        out_specs=[_rows_spec(1024), _qt_spec(), _rows_spec(1024), _rows_spec(1024)],
        out_shape=[_ROWS_T, _QT_T, _ROWS_T, _ROWS_T],
    )(prev, ores, wo, lcs, lcb, lis, lib, wq, wk, wv)


def _tc_upd(prev, ores, wo, lcs, lcb):
    return pl.pallas_call(
        _tc_upd_body, grid=(8,),
        in_specs=[_rows_spec(1024), _rows_spec(1024), _WMAT, _VEC, _VEC],
        out_specs=_rows_spec(1024),
        out_shape=_ROWS_T,
    )(prev, ores, wo, lcs, lcb)


def _tc_gridln(gf, s1, b1, s2, b2):
    return pl.pallas_call(
        _tc_gridln_body, grid=(9,),
        in_specs=[_rows_spec(4096), _VEC, _VEC, _VEC, _VEC],
        out_specs=_rows_spec(4096),
        out_shape=jax.ShapeDtypeStruct((GR, K_DIM), jnp.float32),
    )(gf, s1, b1, s2, b2)


def kernel(grid_scores, N_idx, N_mask, id2lr_pad, S_row_mask, Wq, Wk, Wv, Wo,
           ln_in_scale, ln_in_bias, ln_ch_scale, ln_ch_bias):
    f32 = jnp.float32
    lin = jnp.clip(id2lr_pad[..., 0] * L + id2lr_pad[..., 1], 0, LL - 1).astype(jnp.int32)
    barange = jnp.broadcast_to(jnp.arange(S, dtype=jnp.int32)[None], (B, S))
    wcell = jnp.zeros((B, LL), jnp.int32).at[jnp.arange(B)[:, None], lin].set(barange)
    w_s = jnp.take_along_axis(wcell, lin, axis=1)
    n32 = jnp.clip(N_idx, 0, S - 1).astype(jnp.int32).reshape(BS, KNEI).T
    gidx = (jnp.arange(B, dtype=jnp.int32)[:, None] * LL + lin).reshape(NW, 2, 128)
    wres = (jnp.arange(B, dtype=jnp.int32)[:, None] * S + w_s).reshape(NW, 4, 64)

    gf = grid_scores.reshape(B, K_DIM, LL).transpose(0, 2, 1).reshape(GR, K_DIM)
    lv = [a.reshape(NL, 1, K_DIM).astype(f32)
          for a in (ln_in_scale, ln_in_bias, ln_ch_scale, ln_ch_bias)]
    lis, lib, lcs, lcb = lv

    rows = _sc_gather_rows(gf, gidx, GR)

    def packpairs(x):
        return jax.lax.bitcast_convert_type(
            x.astype(jnp.bfloat16).reshape(BS, K_DIM // 2, 2), jnp.int32)

    q1, k1, v1 = _tc_qkv1(rows, lis[0], lib[0], Wq[0], Wk[0], Wv[0])
    _, ores1 = _sc_attn(q1, packpairs(k1), packpairs(v1), n32, wres)
    rows2, q2, k2, v2 = _tc_upd_qkv(rows, ores1, Wo[0], lcs[0], lcb[0],
                                    lis[1], lib[1], Wq[1], Wk[1], Wv[1])
    _, ores2 = _sc_attn(q2, packpairs(k2), packpairs(v2), n32, wres)
    rows_f = _tc_upd(rows2, ores2, Wo[1], lcs[1], lcb[1])

    ugrid = _tc_gridln(gf, lcs[0], lcb[0], lcs[1], lcb[1])
    out = _sc_assemble(ugrid, rows_f, gidx)

    return out.reshape(B, LL, K_DIM).transpose(0, 2, 1).reshape(B, K_DIM, L, L)
